# baseline (speedup 1.0000x reference)
import functools

import jax
import jax.numpy as jnp
from jax import lax
from jax.experimental import pallas as pl
from jax.experimental.pallas import tpu as pltpu
from jax.experimental.pallas import tpu_sc as plsc

N_NODES = 10000
N_EDGES = 320000
H_DIM = 128
NUM_RELS = 16
NUM_BASES = 4
SUB = H_DIM // NUM_BASES

NC = 2
NS = 16
NBUF = 2
CHUNK = 112
CHUNKS_C0 = 70
CHUNKS_C1 = 110
E_PER_TILE_C0 = CHUNKS_C0 * CHUNK
E_PAD = NS * (CHUNKS_C0 + CHUNKS_C1) * CHUNK
ROWS_PER_TILE = 632
N_PAD = NS * ROWS_PER_TILE


_MM_BLOCK_M = 2000
_MM_GRID_M = N_NODES // _MM_BLOCK_M
_N_Y = NUM_RELS * H_DIM


def _mm_body(x_ref, w_ref, y_ref):
    y_ref[...] = jnp.dot(x_ref[...], w_ref[0],
                         preferred_element_type=jnp.float32)


def _tc_matmul(x, w_r):
    return pl.pallas_call(
        _mm_body,
        grid=(_MM_GRID_M, NUM_RELS),
        in_specs=[
            pl.BlockSpec((_MM_BLOCK_M, H_DIM), lambda i, r: (i, 0)),
            pl.BlockSpec((1, H_DIM, H_DIM), lambda i, r: (r, 0, 0)),
        ],
        out_specs=pl.BlockSpec((_MM_BLOCK_M, H_DIM),
                               lambda i, r: (r * _MM_GRID_M + i, 0)),
        out_shape=jax.ShapeDtypeStruct((NUM_RELS * N_NODES, H_DIM),
                                       jnp.float32),
    )(x, w_r)


def _sc_body(y_hbm, fidx_hbm, dst_hbm, norm_hbm, out_hbm, acc,
             fidx0, fidx1, dst0, dst1, norm0, norm1,
             rows0, rows1, sem0, sem1):
    c = lax.axis_index("c")
    s = lax.axis_index("s")
    row0 = pl.multiple_of(s * ROWS_PER_TILE, 8)
    n_chunks = jnp.where(c == 0, CHUNKS_C0, CHUNKS_C1)
    ebase = jnp.where(
        c == 0, s * (CHUNKS_C0 * CHUNK),
        NS * (CHUNKS_C0 * CHUNK) + s * (CHUNKS_C1 * CHUNK))

    def zrow(i, _):
        for j in range(H_DIM // 16):
            rows0[i, pl.ds(j * 16, 16)] = jnp.zeros((16,), jnp.float32)
        return 0
    lax.fori_loop(0, CHUNK, zrow, 0)
    for k in range(ROWS_PER_TILE // CHUNK):
        pltpu.sync_copy(rows0, acc.at[pl.ds(row0 + k * CHUNK, CHUNK)])
    _rem = ROWS_PER_TILE % CHUNK
    if _rem:
        _done = (ROWS_PER_TILE // CHUNK) * CHUNK
        pltpu.sync_copy(rows0.at[pl.ds(0, _rem)],
                        acc.at[pl.ds(row0 + _done, _rem)])
    plsc.subcore_barrier()

    def scale_chunk(rows_v, norm_v):
        def edge_scale(g, _):
            nv = norm_v[pl.ds(g * 16, 16)]
            for k in range(16):
                e = g * 16 + k
                nb = jnp.full((16,), nv[k], dtype=jnp.float32)
                for j in range(H_DIM // 16):
                    sl = pl.ds(j * 16, 16)
                    rows_v[e, sl] = rows_v[e, sl] * nb
            return 0
        lax.fori_loop(0, CHUNK // 16, edge_scale, 0)

    bufs = ((fidx0, dst0, norm0, rows0, sem0),
            (fidx1, dst1, norm1, rows1, sem1))

    def issue_meta(cidx, bset):
        base = ebase + cidx * CHUNK
        pltpu.async_copy(fidx_hbm.at[pl.ds(base, CHUNK)], bset[0], bset[4])
        pltpu.async_copy(dst_hbm.at[pl.ds(base, CHUNK)], bset[1], bset[4])
        pltpu.async_copy(norm_hbm.at[pl.ds(base, CHUNK)], bset[2], bset[4])

    def wait_meta(cidx, bset):
        base = ebase + cidx * CHUNK
        pltpu.make_async_copy(
            fidx_hbm.at[pl.ds(base, CHUNK)], bset[0], bset[4]).wait()
        pltpu.make_async_copy(
            dst_hbm.at[pl.ds(base, CHUNK)], bset[1], bset[4]).wait()
        pltpu.make_async_copy(
            norm_hbm.at[pl.ds(base, CHUNK)], bset[2], bset[4]).wait()

    def fire(bset):
        pltpu.async_copy(y_hbm.at[bset[0]], bset[3], bset[4])

    issue_meta(0, bufs[0])
    wait_meta(0, bufs[0])
    fire(bufs[0])
    issue_meta(1, bufs[1])

    @pl.loop(0, n_chunks, step=NBUF)
    def chunk_pair(g):
        for b in range(NBUF):
            cidx = g + b
            bset = bufs[b]
            nset = bufs[1 - b]

            def fire_next(cn=cidx + 1, nb=nset):
                wait_meta(cn, nb)
                fire(nb)

            if b == 0:
                fire_next()
            else:
                pl.when(cidx + 1 < n_chunks)(fire_next)

            pltpu.make_async_copy(y_hbm.at[bset[0]], bset[3], bset[4]).wait()
            scale_chunk(bset[3], bset[2])
            pltpu.sync_copy(bset[3], acc.at[bset[1]], add=True)

            @pl.when(cidx + NBUF < n_chunks)
            def _():
                issue_meta(cidx + NBUF, bset)

    plsc.subcore_barrier()
    pltpu.sync_copy(acc.at[pl.ds(row0, ROWS_PER_TILE)],
                    out_hbm.at[c, pl.ds(row0, ROWS_PER_TILE)])


def _sc_call():
    return functools.partial(
        pl.kernel,
        out_type=jax.ShapeDtypeStruct((NC, N_PAD, H_DIM), jnp.float32),
        mesh=plsc.VectorSubcoreMesh(core_axis_name="c", subcore_axis_name="s",
                                    num_cores=NC, num_subcores=NS),
        scratch_types=(
            [pltpu.VMEM_SHARED((N_PAD, H_DIM), jnp.float32)]
            + [pltpu.VMEM((CHUNK,), jnp.int32) for _ in range(2 * NBUF)]
            + [pltpu.VMEM((CHUNK,), jnp.float32) for _ in range(NBUF)]
            + [pltpu.VMEM((CHUNK, H_DIM), jnp.float32) for _ in range(NBUF)]
            + [pltpu.SemaphoreType.DMA for _ in range(NBUF)]
        ),
    )


def _add_body(t_ref, x_ref, lw_ref, bias_ref, o_ref):
    o_ref[...] = (t_ref[0] + t_ref[1] + bias_ref[...]
                  + jnp.dot(x_ref[...], lw_ref[...],
                            preferred_element_type=jnp.float32))


def _tc_combine(out_t, x, lw, bias_row):
    return pl.pallas_call(
        _add_body,
        grid=(_MM_GRID_M,),
        in_specs=[
            pl.BlockSpec((NC, _MM_BLOCK_M, H_DIM), lambda i: (0, i, 0)),
            pl.BlockSpec((_MM_BLOCK_M, H_DIM), lambda i: (i, 0)),
            pl.BlockSpec((H_DIM, H_DIM), lambda i: (0, 0)),
            pl.BlockSpec((1, H_DIM), lambda i: (0, 0)),
        ],
        out_specs=pl.BlockSpec((_MM_BLOCK_M, H_DIM), lambda i: (i, 0)),
        out_shape=jax.ShapeDtypeStruct((N_NODES, H_DIM), jnp.float32),
    )(out_t, x, lw, bias_row)


def kernel(p_feats, edge_index, etype, norm, W_bdd, loop_weight, bias,
           w_relation):
    del w_relation
    x = p_feats.astype(jnp.float32)

    eye = jnp.eye(NUM_BASES, dtype=jnp.float32)
    w_r = jnp.einsum('rbij,bc->rbicj', W_bdd.astype(jnp.float32), eye)
    w_r = w_r.reshape(NUM_RELS, H_DIM, H_DIM)
    bias_row = bias.astype(jnp.float32).reshape(1, H_DIM)

    y = _tc_matmul(x, w_r)

    src = edge_index[0].astype(jnp.int32)
    dst = edge_index[1].astype(jnp.int32)
    fidx = etype.astype(jnp.int32) * N_NODES + src
    pad = E_PAD - N_EDGES
    fidx = jnp.pad(fidx, (0, pad))
    dst_p = jnp.pad(dst, (0, pad))
    norm_p = jnp.pad(norm.astype(jnp.float32).reshape(-1), (0, pad))

    out_t = _sc_call()(_sc_body)(y, fidx, dst_p, norm_p)
    return _tc_combine(out_t, x, loop_weight.astype(jnp.float32), bias_row)


# speedup vs baseline: 1.1174x; 1.1174x over previous
import functools

import jax
import jax.numpy as jnp
from jax import lax
from jax.experimental import pallas as pl
from jax.experimental.pallas import tpu as pltpu
from jax.experimental.pallas import tpu_sc as plsc

N_NODES = 10000
N_EDGES = 320000
H_DIM = 128
NUM_RELS = 16
NUM_BASES = 4
SUB = H_DIM // NUM_BASES

NC = 2
NS = 16
NBUF = 2
CHUNK = 112
CHUNKS_C0 = 110
CHUNKS_C1 = 70
E_PER_TILE_C0 = CHUNKS_C0 * CHUNK
E_PAD = NS * (CHUNKS_C0 + CHUNKS_C1) * CHUNK
ROWS_PER_TILE = 632
N_PAD = NS * ROWS_PER_TILE


_MM_BLOCK_M = 2000
_MM_GRID_M = N_NODES // _MM_BLOCK_M
_N_Y = NUM_RELS * H_DIM


def _mm_body(x_ref, w_ref, y_ref):
    y_ref[...] = jnp.dot(x_ref[...], w_ref[0],
                         preferred_element_type=jnp.float32)


def _tc_matmul(x, w_r):
    return pl.pallas_call(
        _mm_body,
        grid=(_MM_GRID_M, NUM_RELS),
        in_specs=[
            pl.BlockSpec((_MM_BLOCK_M, H_DIM), lambda i, r: (i, 0)),
            pl.BlockSpec((1, H_DIM, H_DIM), lambda i, r: (r, 0, 0)),
        ],
        out_specs=pl.BlockSpec((_MM_BLOCK_M, H_DIM),
                               lambda i, r: (r * _MM_GRID_M + i, 0)),
        out_shape=jax.ShapeDtypeStruct((NUM_RELS * N_NODES, H_DIM),
                                       jnp.float32),
    )(x, w_r)


def _sc_body(y_hbm, fidx_hbm, dst_hbm, norm_hbm, out_hbm, acc,
             fidx0, fidx1, dst0, dst1, norm0, norm1,
             rows0, rows1, sem0, sem1):
    c = lax.axis_index("c")
    s = lax.axis_index("s")
    row0 = pl.multiple_of(s * ROWS_PER_TILE, 8)
    n_chunks = jnp.where(c == 0, CHUNKS_C0, CHUNKS_C1)
    ebase = jnp.where(
        c == 0, s * (CHUNKS_C0 * CHUNK),
        NS * (CHUNKS_C0 * CHUNK) + s * (CHUNKS_C1 * CHUNK))

    def zrow(i, _):
        for j in range(H_DIM // 16):
            rows0[i, pl.ds(j * 16, 16)] = jnp.zeros((16,), jnp.float32)
        return 0
    lax.fori_loop(0, CHUNK, zrow, 0)
    for k in range(ROWS_PER_TILE // CHUNK):
        pltpu.sync_copy(rows0, acc.at[pl.ds(row0 + k * CHUNK, CHUNK)])
    _rem = ROWS_PER_TILE % CHUNK
    if _rem:
        _done = (ROWS_PER_TILE // CHUNK) * CHUNK
        pltpu.sync_copy(rows0.at[pl.ds(0, _rem)],
                        acc.at[pl.ds(row0 + _done, _rem)])
    plsc.subcore_barrier()

    def scale_chunk(rows_v, norm_v):
        def edge_scale(g, _):
            nv = norm_v[pl.ds(g * 16, 16)]
            for k in range(16):
                e = g * 16 + k
                nb = jnp.full((16,), nv[k], dtype=jnp.float32)
                for j in range(H_DIM // 16):
                    sl = pl.ds(j * 16, 16)
                    rows_v[e, sl] = rows_v[e, sl] * nb
            return 0
        lax.fori_loop(0, CHUNK // 16, edge_scale, 0)

    bufs = ((fidx0, dst0, norm0, rows0, sem0),
            (fidx1, dst1, norm1, rows1, sem1))

    def issue_meta(cidx, bset):
        base = ebase + cidx * CHUNK
        pltpu.async_copy(fidx_hbm.at[pl.ds(base, CHUNK)], bset[0], bset[4])
        pltpu.async_copy(dst_hbm.at[pl.ds(base, CHUNK)], bset[1], bset[4])
        pltpu.async_copy(norm_hbm.at[pl.ds(base, CHUNK)], bset[2], bset[4])

    def wait_meta(cidx, bset):
        base = ebase + cidx * CHUNK
        pltpu.make_async_copy(
            fidx_hbm.at[pl.ds(base, CHUNK)], bset[0], bset[4]).wait()
        pltpu.make_async_copy(
            dst_hbm.at[pl.ds(base, CHUNK)], bset[1], bset[4]).wait()
        pltpu.make_async_copy(
            norm_hbm.at[pl.ds(base, CHUNK)], bset[2], bset[4]).wait()

    def fire(bset):
        pltpu.async_copy(y_hbm.at[bset[0]], bset[3], bset[4])

    issue_meta(0, bufs[0])
    wait_meta(0, bufs[0])
    fire(bufs[0])
    issue_meta(1, bufs[1])

    @pl.loop(0, n_chunks, step=NBUF)
    def chunk_pair(g):
        for b in range(NBUF):
            cidx = g + b
            bset = bufs[b]
            nset = bufs[1 - b]

            def fire_next(cn=cidx + 1, nb=nset):
                wait_meta(cn, nb)
                fire(nb)

            if b == 0:
                fire_next()
            else:
                pl.when(cidx + 1 < n_chunks)(fire_next)

            pltpu.make_async_copy(y_hbm.at[bset[0]], bset[3], bset[4]).wait()
            scale_chunk(bset[3], bset[2])
            pltpu.sync_copy(bset[3], acc.at[bset[1]], add=True)

            @pl.when(cidx + NBUF < n_chunks)
            def _():
                issue_meta(cidx + NBUF, bset)

    plsc.subcore_barrier()
    pltpu.sync_copy(acc.at[pl.ds(row0, ROWS_PER_TILE)],
                    out_hbm.at[c, pl.ds(row0, ROWS_PER_TILE)])


def _sc_call():
    return functools.partial(
        pl.kernel,
        out_type=jax.ShapeDtypeStruct((NC, N_PAD, H_DIM), jnp.float32),
        mesh=plsc.VectorSubcoreMesh(core_axis_name="c", subcore_axis_name="s",
                                    num_cores=NC, num_subcores=NS),
        scratch_types=(
            [pltpu.VMEM_SHARED((N_PAD, H_DIM), jnp.float32)]
            + [pltpu.VMEM((CHUNK,), jnp.int32) for _ in range(2 * NBUF)]
            + [pltpu.VMEM((CHUNK,), jnp.float32) for _ in range(NBUF)]
            + [pltpu.VMEM((CHUNK, H_DIM), jnp.float32) for _ in range(NBUF)]
            + [pltpu.SemaphoreType.DMA for _ in range(NBUF)]
        ),
    )


def _add_body(t_ref, x_ref, lw_ref, bias_ref, o_ref):
    o_ref[...] = (t_ref[0] + t_ref[1] + bias_ref[...]
                  + jnp.dot(x_ref[...], lw_ref[...],
                            preferred_element_type=jnp.float32))


def _tc_combine(out_t, x, lw, bias_row):
    return pl.pallas_call(
        _add_body,
        grid=(_MM_GRID_M,),
        in_specs=[
            pl.BlockSpec((NC, _MM_BLOCK_M, H_DIM), lambda i: (0, i, 0)),
            pl.BlockSpec((_MM_BLOCK_M, H_DIM), lambda i: (i, 0)),
            pl.BlockSpec((H_DIM, H_DIM), lambda i: (0, 0)),
            pl.BlockSpec((1, H_DIM), lambda i: (0, 0)),
        ],
        out_specs=pl.BlockSpec((_MM_BLOCK_M, H_DIM), lambda i: (i, 0)),
        out_shape=jax.ShapeDtypeStruct((N_NODES, H_DIM), jnp.float32),
    )(out_t, x, lw, bias_row)


def kernel(p_feats, edge_index, etype, norm, W_bdd, loop_weight, bias,
           w_relation):
    del w_relation
    x = p_feats.astype(jnp.float32)

    eye = jnp.eye(NUM_BASES, dtype=jnp.float32)
    w_r = jnp.einsum('rbij,bc->rbicj', W_bdd.astype(jnp.float32), eye)
    w_r = w_r.reshape(NUM_RELS, H_DIM, H_DIM)
    bias_row = bias.astype(jnp.float32).reshape(1, H_DIM)

    y = _tc_matmul(x, w_r)

    src = edge_index[0].astype(jnp.int32)
    dst = edge_index[1].astype(jnp.int32)
    fidx = etype.astype(jnp.int32) * N_NODES + src
    pad = E_PAD - N_EDGES
    fidx = jnp.pad(fidx, (0, pad))
    dst_p = jnp.pad(dst, (0, pad))
    norm_p = jnp.pad(norm.astype(jnp.float32).reshape(-1), (0, pad))

    out_t = _sc_call()(_sc_body)(y, fidx, dst_p, norm_p)
    return _tc_combine(out_t, x, loop_weight.astype(jnp.float32), bias_row)


# speedup vs baseline: 1.2357x; 1.1059x over previous
import functools

import jax
import jax.numpy as jnp
from jax import lax
from jax.experimental import pallas as pl
from jax.experimental.pallas import tpu as pltpu
from jax.experimental.pallas import tpu_sc as plsc

N_NODES = 10000
N_EDGES = 320000
H_DIM = 128
NUM_RELS = 16
NUM_BASES = 4
SUB = H_DIM // NUM_BASES

NC = 2
NS = 16
NBUF = 2
CHUNK = 112
CHUNKS_C0 = 110
CHUNKS_C1 = 70
E_PER_TILE_C0 = CHUNKS_C0 * CHUNK
E_PAD = NS * (CHUNKS_C0 + CHUNKS_C1) * CHUNK
ROWS_PER_TILE = 632
N_PAD = NS * ROWS_PER_TILE


_MM_BLOCK_M = 2000
_MM_GRID_M = N_NODES // _MM_BLOCK_M
_N_Y = NUM_RELS * H_DIM


def _mm_body(x_ref, w_ref, y_ref):
    out = jnp.dot(x_ref[...], w_ref[...], preferred_element_type=jnp.float32)
    for r in range(NUM_RELS):
        y_ref[r, ...] = out[:, r * H_DIM:(r + 1) * H_DIM]


def _tc_matmul(x, w_cat):
    return pl.pallas_call(
        _mm_body,
        grid=(_MM_GRID_M,),
        in_specs=[
            pl.BlockSpec((_MM_BLOCK_M, H_DIM), lambda i: (i, 0)),
            pl.BlockSpec((H_DIM, _N_Y), lambda i: (0, 0)),
        ],
        out_specs=pl.BlockSpec((NUM_RELS, _MM_BLOCK_M, H_DIM),
                               lambda i: (0, i, 0)),
        out_shape=jax.ShapeDtypeStruct((NUM_RELS, N_NODES, H_DIM),
                                       jnp.float32),
    )(x, w_cat)


def _sc_body(y_hbm, fidx_hbm, dst_hbm, norm_hbm, out_hbm, acc,
             fidx0, fidx1, dst0, dst1, norm0, norm1,
             rows0, rows1, sem0, sem1):
    c = lax.axis_index("c")
    s = lax.axis_index("s")
    row0 = pl.multiple_of(s * ROWS_PER_TILE, 8)
    n_chunks = jnp.where(c == 0, CHUNKS_C0, CHUNKS_C1)
    ebase = jnp.where(
        c == 0, s * (CHUNKS_C0 * CHUNK),
        NS * (CHUNKS_C0 * CHUNK) + s * (CHUNKS_C1 * CHUNK))

    def zrow(i, _):
        for j in range(H_DIM // 16):
            rows0[i, pl.ds(j * 16, 16)] = jnp.zeros((16,), jnp.float32)
        return 0
    lax.fori_loop(0, CHUNK, zrow, 0)
    for k in range(ROWS_PER_TILE // CHUNK):
        pltpu.sync_copy(rows0, acc.at[pl.ds(row0 + k * CHUNK, CHUNK)])
    _rem = ROWS_PER_TILE % CHUNK
    if _rem:
        _done = (ROWS_PER_TILE // CHUNK) * CHUNK
        pltpu.sync_copy(rows0.at[pl.ds(0, _rem)],
                        acc.at[pl.ds(row0 + _done, _rem)])
    plsc.subcore_barrier()

    def scale_chunk(rows_v, norm_v):
        def edge_scale(g, _):
            nv = norm_v[pl.ds(g * 16, 16)]
            for k in range(16):
                e = g * 16 + k
                nb = jnp.full((16,), nv[k], dtype=jnp.float32)
                for j in range(H_DIM // 16):
                    sl = pl.ds(j * 16, 16)
                    rows_v[e, sl] = rows_v[e, sl] * nb
            return 0
        lax.fori_loop(0, CHUNK // 16, edge_scale, 0)

    bufs = ((fidx0, dst0, norm0, rows0, sem0),
            (fidx1, dst1, norm1, rows1, sem1))

    def issue_meta(cidx, bset):
        base = ebase + cidx * CHUNK
        pltpu.async_copy(fidx_hbm.at[pl.ds(base, CHUNK)], bset[0], bset[4])
        pltpu.async_copy(dst_hbm.at[pl.ds(base, CHUNK)], bset[1], bset[4])
        pltpu.async_copy(norm_hbm.at[pl.ds(base, CHUNK)], bset[2], bset[4])

    def wait_meta(cidx, bset):
        base = ebase + cidx * CHUNK
        pltpu.make_async_copy(
            fidx_hbm.at[pl.ds(base, CHUNK)], bset[0], bset[4]).wait()
        pltpu.make_async_copy(
            dst_hbm.at[pl.ds(base, CHUNK)], bset[1], bset[4]).wait()
        pltpu.make_async_copy(
            norm_hbm.at[pl.ds(base, CHUNK)], bset[2], bset[4]).wait()

    def fire(bset):
        pltpu.async_copy(y_hbm.at[bset[0]], bset[3], bset[4])

    issue_meta(0, bufs[0])
    wait_meta(0, bufs[0])
    fire(bufs[0])
    issue_meta(1, bufs[1])

    @pl.loop(0, n_chunks, step=NBUF)
    def chunk_pair(g):
        for b in range(NBUF):
            cidx = g + b
            bset = bufs[b]
            nset = bufs[1 - b]

            def fire_next(cn=cidx + 1, nb=nset):
                wait_meta(cn, nb)
                fire(nb)

            if b == 0:
                fire_next()
            else:
                pl.when(cidx + 1 < n_chunks)(fire_next)

            pltpu.make_async_copy(y_hbm.at[bset[0]], bset[3], bset[4]).wait()
            scale_chunk(bset[3], bset[2])
            pltpu.sync_copy(bset[3], acc.at[bset[1]], add=True)

            @pl.when(cidx + NBUF < n_chunks)
            def _():
                issue_meta(cidx + NBUF, bset)

    plsc.subcore_barrier()
    pltpu.sync_copy(acc.at[pl.ds(row0, ROWS_PER_TILE)],
                    out_hbm.at[c, pl.ds(row0, ROWS_PER_TILE)])


def _sc_call():
    return functools.partial(
        pl.kernel,
        out_type=jax.ShapeDtypeStruct((NC, N_PAD, H_DIM), jnp.float32),
        mesh=plsc.VectorSubcoreMesh(core_axis_name="c", subcore_axis_name="s",
                                    num_cores=NC, num_subcores=NS),
        scratch_types=(
            [pltpu.VMEM_SHARED((N_PAD, H_DIM), jnp.float32)]
            + [pltpu.VMEM((CHUNK,), jnp.int32) for _ in range(2 * NBUF)]
            + [pltpu.VMEM((CHUNK,), jnp.float32) for _ in range(NBUF)]
            + [pltpu.VMEM((CHUNK, H_DIM), jnp.float32) for _ in range(NBUF)]
            + [pltpu.SemaphoreType.DMA for _ in range(NBUF)]
        ),
    )


def _add_body(t_ref, x_ref, lw_ref, bias_ref, o_ref):
    o_ref[...] = (t_ref[0] + t_ref[1] + bias_ref[...]
                  + jnp.dot(x_ref[...], lw_ref[...],
                            preferred_element_type=jnp.float32))


def _tc_combine(out_t, x, lw, bias_row):
    return pl.pallas_call(
        _add_body,
        grid=(_MM_GRID_M,),
        in_specs=[
            pl.BlockSpec((NC, _MM_BLOCK_M, H_DIM), lambda i: (0, i, 0)),
            pl.BlockSpec((_MM_BLOCK_M, H_DIM), lambda i: (i, 0)),
            pl.BlockSpec((H_DIM, H_DIM), lambda i: (0, 0)),
            pl.BlockSpec((1, H_DIM), lambda i: (0, 0)),
        ],
        out_specs=pl.BlockSpec((_MM_BLOCK_M, H_DIM), lambda i: (i, 0)),
        out_shape=jax.ShapeDtypeStruct((N_NODES, H_DIM), jnp.float32),
    )(out_t, x, lw, bias_row)


def kernel(p_feats, edge_index, etype, norm, W_bdd, loop_weight, bias,
           w_relation):
    del w_relation
    x = p_feats.astype(jnp.float32)

    eye = jnp.eye(NUM_BASES, dtype=jnp.float32)
    w_full = jnp.einsum('rbij,bc->bircj', W_bdd.astype(jnp.float32), eye)
    w_full = w_full.reshape(H_DIM, _N_Y)
    bias_row = bias.astype(jnp.float32).reshape(1, H_DIM)

    y = _tc_matmul(x, w_full).reshape(NUM_RELS * N_NODES, H_DIM)

    src = edge_index[0].astype(jnp.int32)
    dst = edge_index[1].astype(jnp.int32)
    fidx = etype.astype(jnp.int32) * N_NODES + src
    pad = E_PAD - N_EDGES
    fidx = jnp.pad(fidx, (0, pad))
    dst_p = jnp.pad(dst, (0, pad))
    norm_p = jnp.pad(norm.astype(jnp.float32).reshape(-1), (0, pad))

    out_t = _sc_call()(_sc_body)(y, fidx, dst_p, norm_p)
    return _tc_combine(out_t, x, loop_weight.astype(jnp.float32), bias_row)
